# baseline (speedup 1.0000x reference)
import functools

import jax
import jax.numpy as jnp
from jax import lax
from jax.experimental import pallas as pl
from jax.experimental.pallas import tpu as pltpu
from jax.experimental.pallas import tpu_sc as plsc

_D = 2048
_R = 16
_P = 3
_PHI = 0.1
_TAU = 0.1
_SCALE = 1.0
_BATCH = 8192
_KAPPA = max(1, _R // _P)
_E = _P + 1
_RT = _E * _R
_BR1 = 2048
_NB1 = _BATCH // _BR1
_BR2 = 1024
_NB2 = _BATCH // _BR2


def _zpass_body(x_ref, a_ref, z_ref, zmean_ref):
    b = pl.program_id(0)
    z = lax.dot_general(
        x_ref[...], a_ref[...], (((1,), (1,)), ((), ())),
        preferred_element_type=jnp.float32)
    z_ref[...] = z.astype(jnp.bfloat16)
    part = jnp.sum(jnp.abs(z), axis=0, keepdims=True) * (1.0 / _BATCH)

    @pl.when(b == 0)
    def _():
        zmean_ref[...] = jnp.zeros_like(zmean_ref)

    zmean_ref[...] += part


_zpass = pl.pallas_call(
    _zpass_body,
    grid=(_NB1,),
    in_specs=[
        pl.BlockSpec((_BR1, _D), lambda b: (b, 0)),
        pl.BlockSpec((_RT, _D), lambda b: (0, 0)),
    ],
    out_specs=[
        pl.BlockSpec((_BR1, _RT), lambda b: (b, 0)),
        pl.BlockSpec((1, _RT), lambda b: (0, 0)),
    ],
    out_shape=[
        jax.ShapeDtypeStruct((_BATCH, _RT), jnp.bfloat16),
        jax.ShapeDtypeStruct((1, _RT), jnp.float32),
    ],
    compiler_params=pltpu.CompilerParams(skip_device_barrier=True),
)


@functools.cache
def _build_select_mask_sc():
    @functools.partial(
        pl.kernel,
        mesh=plsc.VectorSubcoreMesh(core_axis_name="c", subcore_axis_name="s", num_cores=1, num_subcores=1),
        out_type=jax.ShapeDtypeStruct((_E, _R), jnp.float32),
        scratch_types=[
            pltpu.VMEM((_E, _R), jnp.float32),
            pltpu.VMEM((_E, _R), jnp.float32),
        ],
        compiler_params=pltpu.CompilerParams(
            needs_layout_passes=False, skip_device_barrier=True),
    )
    def _select_mask_sc(zmean_hbm, mask_hbm, zm_v, mask_v):
        c = lax.axis_index("c")
        s = lax.axis_index("s")

        @pl.when(jnp.logical_and(c == 0, s == 0))
        def _():
            pltpu.sync_copy(zmean_hbm, zm_v)
            idx = lax.iota(jnp.int32, _R)
            for e in range(_E):
                thr = _PHI if e < _P else _TAU
                v = zm_v[e]
                _, perm = plsc.sort_key_val(-v, idx)
                n_active = plsc.all_reduce_population_count(v > thr)
                count = jnp.minimum(jnp.maximum(n_active, 1), _KAPPA)
                sel = jnp.where(idx < count, 1.0, 0.0).astype(jnp.float32)
                _, mask_row = plsc.sort_key_val(perm, sel)
                mask_v[e] = mask_row
            pltpu.sync_copy(mask_v, mask_hbm)

    return _select_mask_sc


def _outpass_body(z_ref, m_ref, w0_ref, bt_ref, o_ref):
    zm = z_ref[...].astype(jnp.float32) * m_ref[...]
    r = lax.dot_general(
        zm, bt_ref[...], (((1,), (0,)), ((), ())),
        preferred_element_type=jnp.float32)
    o_ref[...] = w0_ref[...] + (_SCALE / _R) * r


_outpass = pl.pallas_call(
    _outpass_body,
    grid=(_NB2,),
    in_specs=[
        pl.BlockSpec((_BR2, _RT), lambda b: (b, 0)),
        pl.BlockSpec((1, _RT), lambda b: (0, 0)),
        pl.BlockSpec((_BR2, _D), lambda b: (b, 0)),
        pl.BlockSpec((_RT, _D), lambda b: (0, 0)),
    ],
    out_specs=pl.BlockSpec((_BR2, _D), lambda b: (b, 0)),
    out_shape=jax.ShapeDtypeStruct((_BATCH, _D), jnp.float32),
    compiler_params=pltpu.CompilerParams(
        skip_device_barrier=True, dimension_semantics=("parallel",)),
)


def kernel(x, W0_output, A_experts, B_experts, A_g, B_g):
    A_all = jnp.concatenate([A_experts.reshape(_P * _R, _D), A_g], axis=0)
    B_allT = jnp.concatenate(
        [B_experts.transpose(0, 2, 1).reshape(_P * _R, _D), B_g.T], axis=0)
    Z, z_mean = _zpass(x, A_all)
    mask = _build_select_mask_sc()(z_mean.reshape(_E, _R))
    return _outpass(Z, mask.reshape(1, _RT), W0_output, B_allT)


# speedup vs baseline: 1.0159x; 1.0159x over previous
import functools

import jax
import jax.numpy as jnp
from jax import lax
from jax.experimental import pallas as pl
from jax.experimental.pallas import tpu as pltpu
from jax.experimental.pallas import tpu_sc as plsc

_D = 2048
_R = 16
_P = 3
_PHI = 0.1
_TAU = 0.1
_SCALE = 1.0
_BATCH = 8192
_KAPPA = max(1, _R // _P)
_E = _P + 1
_RT = _E * _R
_BR1 = 1024
_NB1 = _BATCH // _BR1
_BR2 = 1024
_NB2 = _BATCH // _BR2


def _zpass_body(x_ref, a_ref, z_ref, zmean_ref):
    b = pl.program_id(0)
    z = lax.dot_general(
        x_ref[...].astype(jnp.bfloat16), a_ref[...].astype(jnp.bfloat16),
        (((1,), (1,)), ((), ())),
        preferred_element_type=jnp.float32)
    z_ref[...] = z.astype(jnp.bfloat16)
    part = jnp.sum(jnp.abs(z), axis=0, keepdims=True) * (1.0 / _BATCH)

    @pl.when(b == 0)
    def _():
        zmean_ref[...] = jnp.zeros_like(zmean_ref)

    zmean_ref[...] += part


_zpass = pl.pallas_call(
    _zpass_body,
    grid=(_NB1,),
    in_specs=[
        pl.BlockSpec((_BR1, _D), lambda b: (b, 0)),
        pl.BlockSpec((_RT, _D), lambda b: (0, 0)),
    ],
    out_specs=[
        pl.BlockSpec((_BR1, _RT), lambda b: (b, 0)),
        pl.BlockSpec((1, _RT), lambda b: (0, 0)),
    ],
    out_shape=[
        jax.ShapeDtypeStruct((_BATCH, _RT), jnp.bfloat16),
        jax.ShapeDtypeStruct((1, _RT), jnp.float32),
    ],
    compiler_params=pltpu.CompilerParams(skip_device_barrier=True),
)


@functools.cache
def _build_select_mask_sc():
    @functools.partial(
        pl.kernel,
        mesh=plsc.VectorSubcoreMesh(core_axis_name="c", subcore_axis_name="s", num_cores=1, num_subcores=1),
        out_type=jax.ShapeDtypeStruct((_E, _R), jnp.float32),
        scratch_types=[
            pltpu.VMEM((_E, _R), jnp.float32),
            pltpu.VMEM((_E, _R), jnp.float32),
        ],
        compiler_params=pltpu.CompilerParams(
            needs_layout_passes=False, skip_device_barrier=True),
    )
    def _select_mask_sc(zmean_hbm, mask_hbm, zm_v, mask_v):
        c = lax.axis_index("c")
        s = lax.axis_index("s")

        @pl.when(jnp.logical_and(c == 0, s == 0))
        def _():
            pltpu.sync_copy(zmean_hbm, zm_v)
            idx = lax.iota(jnp.int32, _R)
            for e in range(_E):
                thr = _PHI if e < _P else _TAU
                v = zm_v[e]
                _, perm = plsc.sort_key_val(-v, idx)
                n_active = plsc.all_reduce_population_count(v > thr)
                count = jnp.minimum(jnp.maximum(n_active, 1), _KAPPA)
                sel = jnp.where(idx < count, 1.0, 0.0).astype(jnp.float32)
                _, mask_row = plsc.sort_key_val(perm, sel)
                mask_v[e] = mask_row
            pltpu.sync_copy(mask_v, mask_hbm)

    return _select_mask_sc


def _outpass_body(z_ref, m_ref, w0_ref, bt_ref, o_ref):
    zm = z_ref[...] * m_ref[...].astype(jnp.bfloat16)
    r = lax.dot_general(
        zm, bt_ref[...].astype(jnp.bfloat16), (((1,), (0,)), ((), ())),
        preferred_element_type=jnp.float32)
    o_ref[...] = w0_ref[...] + (_SCALE / _R) * r


_outpass = pl.pallas_call(
    _outpass_body,
    grid=(_NB2,),
    in_specs=[
        pl.BlockSpec((_BR2, _RT), lambda b: (b, 0)),
        pl.BlockSpec((1, _RT), lambda b: (0, 0)),
        pl.BlockSpec((_BR2, _D), lambda b: (b, 0)),
        pl.BlockSpec((_RT, _D), lambda b: (0, 0)),
    ],
    out_specs=pl.BlockSpec((_BR2, _D), lambda b: (b, 0)),
    out_shape=jax.ShapeDtypeStruct((_BATCH, _D), jnp.float32),
    compiler_params=pltpu.CompilerParams(
        skip_device_barrier=True, dimension_semantics=("parallel",)),
)


def kernel(x, W0_output, A_experts, B_experts, A_g, B_g):
    A_all = jnp.concatenate([A_experts.reshape(_P * _R, _D), A_g], axis=0)
    B_allT = jnp.concatenate(
        [B_experts.transpose(0, 2, 1).reshape(_P * _R, _D), B_g.T], axis=0)
    Z, z_mean = _zpass(x, A_all)
    mask = _build_select_mask_sc()(z_mean.reshape(_E, _R))
    return _outpass(Z, mask.reshape(1, _RT), W0_output, B_allT)


# speedup vs baseline: 1.0468x; 1.0305x over previous
import functools

import jax
import jax.numpy as jnp
from jax import lax
from jax.experimental import pallas as pl
from jax.experimental.pallas import tpu as pltpu
from jax.experimental.pallas import tpu_sc as plsc

_D = 2048
_R = 16
_P = 3
_PHI = 0.1
_TAU = 0.1
_SCALE = 1.0
_BATCH = 8192
_KAPPA = max(1, _R // _P)
_E = _P + 1
_RT = _E * _R
_BR1 = 1024
_NB1 = _BATCH // _BR1
_BR2 = 1024
_NB2 = _BATCH // _BR2


def _zpass_body(x_ref, a_ref, z_ref, zmean_ref):
    b = pl.program_id(0)
    z = lax.dot_general(
        x_ref[...], a_ref[...], (((1,), (1,)), ((), ())),
        preferred_element_type=jnp.float32)
    z_ref[...] = z.astype(jnp.bfloat16)
    part = jnp.sum(jnp.abs(z), axis=0, keepdims=True) * (1.0 / _BATCH)

    @pl.when(b == 0)
    def _():
        zmean_ref[...] = jnp.zeros_like(zmean_ref)

    zmean_ref[...] += part


_zpass = pl.pallas_call(
    _zpass_body,
    grid=(_NB1,),
    in_specs=[
        pl.BlockSpec((_BR1, _D), lambda b: (b, 0)),
        pl.BlockSpec((_RT, _D), lambda b: (0, 0)),
    ],
    out_specs=[
        pl.BlockSpec((_BR1, _RT), lambda b: (b, 0)),
        pl.BlockSpec((1, _RT), lambda b: (0, 0)),
    ],
    out_shape=[
        jax.ShapeDtypeStruct((_BATCH, _RT), jnp.bfloat16),
        jax.ShapeDtypeStruct((1, _RT), jnp.float32),
    ],
    compiler_params=pltpu.CompilerParams(
        skip_device_barrier=True, allow_input_fusion=[False, True]),
)


@functools.cache
def _build_select_mask_sc():
    @functools.partial(
        pl.kernel,
        mesh=plsc.VectorSubcoreMesh(core_axis_name="c", subcore_axis_name="s", num_cores=1, num_subcores=1),
        out_type=jax.ShapeDtypeStruct((_E, _R), jnp.float32),
        scratch_types=[
            pltpu.VMEM((_E, _R), jnp.float32),
            pltpu.VMEM((_E, _R), jnp.float32),
        ],
        compiler_params=pltpu.CompilerParams(
            needs_layout_passes=False, skip_device_barrier=True),
    )
    def _select_mask_sc(zmean_hbm, mask_hbm, zm_v, mask_v):
        c = lax.axis_index("c")
        s = lax.axis_index("s")

        @pl.when(jnp.logical_and(c == 0, s == 0))
        def _():
            pltpu.sync_copy(zmean_hbm, zm_v)
            idx = lax.iota(jnp.int32, _R)
            for e in range(_E):
                thr = _PHI if e < _P else _TAU
                v = zm_v[e]
                _, perm = plsc.sort_key_val(-v, idx)
                n_active = plsc.all_reduce_population_count(v > thr)
                count = jnp.minimum(jnp.maximum(n_active, 1), _KAPPA)
                sel = jnp.where(idx < count, 1.0, 0.0).astype(jnp.float32)
                _, mask_row = plsc.sort_key_val(perm, sel)
                mask_v[e] = mask_row
            pltpu.sync_copy(mask_v, mask_hbm)

    return _select_mask_sc


def _outpass_body(z_ref, m_ref, w0_ref, bt_ref, o_ref):
    zm = z_ref[...].astype(jnp.float32) * m_ref[...]
    r = lax.dot_general(
        zm, bt_ref[...], (((1,), (0,)), ((), ())),
        preferred_element_type=jnp.float32)
    o_ref[...] = w0_ref[...] + (_SCALE / _R) * r


_outpass = pl.pallas_call(
    _outpass_body,
    grid=(_NB2,),
    in_specs=[
        pl.BlockSpec((_BR2, _RT), lambda b: (b, 0)),
        pl.BlockSpec((1, _RT), lambda b: (0, 0)),
        pl.BlockSpec((_BR2, _D), lambda b: (b, 0)),
        pl.BlockSpec((_RT, _D), lambda b: (0, 0)),
    ],
    out_specs=pl.BlockSpec((_BR2, _D), lambda b: (b, 0)),
    out_shape=jax.ShapeDtypeStruct((_BATCH, _D), jnp.float32),
    compiler_params=pltpu.CompilerParams(
        skip_device_barrier=True, dimension_semantics=("parallel",),
        allow_input_fusion=[False, False, False, True]),
)


def kernel(x, W0_output, A_experts, B_experts, A_g, B_g):
    A_all = jnp.concatenate([A_experts.reshape(_P * _R, _D), A_g], axis=0)
    B_allT = jnp.concatenate(
        [B_experts.transpose(0, 2, 1).reshape(_P * _R, _D), B_g.T], axis=0)
    Z, z_mean = _zpass(x, A_all)
    mask = _build_select_mask_sc()(z_mean.reshape(_E, _R))
    return _outpass(Z, mask.reshape(1, _RT), W0_output, B_allT)


# speedup vs baseline: 1.0883x; 1.0397x over previous
import functools

import jax
import jax.numpy as jnp
from jax import lax
from jax.experimental import pallas as pl
from jax.experimental.pallas import tpu as pltpu
from jax.experimental.pallas import tpu_sc as plsc

_D = 2048
_R = 16
_P = 3
_PHI = 0.1
_TAU = 0.1
_SCALE = 1.0
_BATCH = 8192
_KAPPA = max(1, _R // _P)
_E = _P + 1
_RT = _E * _R
_BR1 = 1024
_NB1 = _BATCH // _BR1
_BR2 = 1024
_NB2 = _BATCH // _BR2


def _zpass_body(x_ref, a_ref, z_ref, zmean_ref):
    b = pl.program_id(0)
    z = lax.dot_general(
        x_ref[...], a_ref[...], (((1,), (1,)), ((), ())),
        preferred_element_type=jnp.float32)
    z_ref[...] = z.astype(jnp.bfloat16)
    part = jnp.sum(jnp.abs(z), axis=0, keepdims=True) * (1.0 / _BATCH)

    @pl.when(b == 0)
    def _():
        zmean_ref[...] = jnp.zeros_like(zmean_ref)

    zmean_ref[...] += part


_zpass = pl.pallas_call(
    _zpass_body,
    grid=(_NB1,),
    in_specs=[
        pl.BlockSpec((_BR1, _D), lambda b: (b, 0)),
        pl.BlockSpec((_RT, _D), lambda b: (0, 0)),
    ],
    out_specs=[
        pl.BlockSpec((_BR1, _RT), lambda b: (b, 0)),
        pl.BlockSpec((1, _RT), lambda b: (0, 0)),
    ],
    out_shape=[
        jax.ShapeDtypeStruct((_BATCH, _RT), jnp.bfloat16),
        jax.ShapeDtypeStruct((1, _RT), jnp.float32),
    ],
    compiler_params=pltpu.CompilerParams(
        skip_device_barrier=True, allow_input_fusion=[False, True]),
)


@functools.cache
def _build_select_mask_sc():
    @functools.partial(
        pl.kernel,
        mesh=plsc.VectorSubcoreMesh(core_axis_name="c", subcore_axis_name="s", num_cores=1, num_subcores=1),
        out_type=jax.ShapeDtypeStruct((1, _RT), jnp.float32),
        scratch_types=[
            pltpu.VMEM((1, _RT), jnp.float32),
            pltpu.VMEM((1, _RT), jnp.float32),
        ],
        compiler_params=pltpu.CompilerParams(
            needs_layout_passes=False, skip_device_barrier=True),
    )
    def _select_mask_sc(zmean_hbm, mask_hbm, zm_v, mask_v):
        c = lax.axis_index("c")
        s = lax.axis_index("s")

        @pl.when(jnp.logical_and(c == 0, s == 0))
        def _():
            pltpu.sync_copy(zmean_hbm, zm_v)
            idx = lax.iota(jnp.int32, _R)
            for e in range(_E):
                thr = _PHI if e < _P else _TAU
                v = zm_v[0, pl.ds(e * _R, _R)]
                _, perm = plsc.sort_key_val(-v, idx)
                n_active = plsc.all_reduce_population_count(v > thr)
                count = jnp.minimum(jnp.maximum(n_active, 1), _KAPPA)
                sel = jnp.where(idx < count, 1.0, 0.0).astype(jnp.float32)
                _, mask_row = plsc.sort_key_val(perm, sel)
                mask_v[0, pl.ds(e * _R, _R)] = mask_row
            pltpu.sync_copy(mask_v, mask_hbm)

    return _select_mask_sc


def _outpass_body(z_ref, m_ref, w0_ref, bt_ref, o_ref):
    zm = z_ref[...].astype(jnp.float32) * m_ref[...]
    r = lax.dot_general(
        zm, bt_ref[...], (((1,), (0,)), ((), ())),
        preferred_element_type=jnp.float32)
    o_ref[...] = w0_ref[...] + (_SCALE / _R) * r


_outpass = pl.pallas_call(
    _outpass_body,
    grid=(_NB2,),
    in_specs=[
        pl.BlockSpec((_BR2, _RT), lambda b: (b, 0)),
        pl.BlockSpec((1, _RT), lambda b: (0, 0)),
        pl.BlockSpec((_BR2, _D), lambda b: (b, 0)),
        pl.BlockSpec((_RT, _D), lambda b: (0, 0)),
    ],
    out_specs=pl.BlockSpec((_BR2, _D), lambda b: (b, 0)),
    out_shape=jax.ShapeDtypeStruct((_BATCH, _D), jnp.float32),
    compiler_params=pltpu.CompilerParams(
        skip_device_barrier=True, dimension_semantics=("parallel",),
        allow_input_fusion=[False, False, False, True]),
)


def kernel(x, W0_output, A_experts, B_experts, A_g, B_g):
    A_all = jnp.concatenate([A_experts.reshape(_P * _R, _D), A_g], axis=0)
    B_allT = jnp.concatenate(
        [B_experts.transpose(0, 2, 1).reshape(_P * _R, _D), B_g.T], axis=0)
    Z, z_mean = _zpass(x, A_all)
    mask = _build_select_mask_sc()(z_mean)
    return _outpass(Z, mask, W0_output, B_allT)
